# baseline (speedup 1.0000x reference)
import functools
import math

import jax
import jax.numpy as jnp
from jax import lax
from jax.experimental import pallas as pl
from jax.experimental.pallas import tpu as pltpu
from jax.experimental.pallas import tpu_sc as plsc

N = 4096
D = 512
H = 4
HD = D // H
E = 131072
BQ = 256
BKV = 256
NEG = -1e30

SC_CORES = 2
SC_TILES = 16
EPW = E // (SC_CORES * SC_TILES)


def _scatter_body(edge_ref, ones_ref, mask_ref, rbuf, cbuf, idx1d, ones1d,
                  sem, sem2):
    core = lax.axis_index("c")
    sub = lax.axis_index("s")
    wid = sub * SC_CORES + core
    h_r = pltpu.async_copy(edge_ref.at[pl.ds(wid * EPW, EPW)], rbuf, sem2)
    h_c = pltpu.async_copy(edge_ref.at[pl.ds(E + wid * EPW, EPW)], cbuf, sem2)
    pltpu.sync_copy(ones_ref, ones1d)
    h_r.wait()
    h_c.wait()

    def compute_row(j, carry):
        for i in range(8):
            off = j * 128 + i * 16
            rv = rbuf[pl.ds(off, 16)]
            cv = cbuf[pl.ds(off, 16)]
            idx1d[pl.ds(off, 16)] = rv * N + cv
        return carry

    lax.fori_loop(0, EPW // 128, compute_row, 0)

    pltpu.async_copy(ones1d, mask_ref.at[idx1d], sem).wait()


def _build_mask(edge_flat, ones_arr, mask_ref):
    mesh = plsc.VectorSubcoreMesh(core_axis_name="c", subcore_axis_name="s",
                                  num_cores=SC_CORES)
    f = pl.kernel(
        _scatter_body,
        mesh=mesh,
        out_type=(),
        scratch_types=[
            pltpu.VMEM((EPW,), jnp.int32),
            pltpu.VMEM((EPW,), jnp.int32),
            pltpu.VMEM((EPW,), jnp.int32),
            pltpu.VMEM((EPW,), jnp.float32),
            pltpu.SemaphoreType.DMA,
            pltpu.SemaphoreType.DMA,
        ],
    )
    return f(edge_flat, ones_arr, mask_ref)


def _kv_proj_body(x_ref, wk_ref, bk_ref, wv_ref, bv_ref, k_ref, v_ref):
    x = x_ref[...]
    dn = (((1,), (1,)), ((), ()))
    k = lax.dot_general(x, wk_ref[...], dn,
                        preferred_element_type=jnp.float32) + bk_ref[...]
    v = lax.dot_general(x, wv_ref[...], dn,
                        preferred_element_type=jnp.float32) + bv_ref[...]
    k_ref[...] = k.astype(jnp.bfloat16)
    v_ref[...] = v.astype(jnp.bfloat16)


def _attn_body(x_ref, wq_ref, bq_ref, k_ref, v_ref, mask_ref, wo_ref, bo_ref,
               out_ref):
    dn = (((1,), (1,)), ((), ()))
    x = x_ref[...].astype(jnp.bfloat16)
    q = lax.dot_general(x, wq_ref[...].astype(jnp.bfloat16), dn,
                        preferred_element_type=jnp.float32) + bq_ref[...]
    q = q * (1.0 / math.sqrt(HD))
    bias = mask_ref[...]
    valid = jnp.max(bias, axis=1, keepdims=True) > -0.5e30
    heads = []
    for h in range(H):
        sl = slice(h * HD, (h + 1) * HD)
        qh = q[:, sl].astype(jnp.bfloat16)
        kh = k_ref[:, sl]
        s = lax.dot_general(qh, kh, dn, preferred_element_type=jnp.float32)
        s = s + bias
        m = jnp.max(s, axis=1, keepdims=True)
        e = jnp.exp(s - m)
        l = jnp.sum(e, axis=1, keepdims=True)
        acc = jnp.dot(e.astype(jnp.bfloat16), v_ref[:, sl],
                      preferred_element_type=jnp.float32)
        heads.append(jnp.where(valid, acc / jnp.maximum(l, 1e-30), 0.0))
    att = jnp.concatenate(heads, axis=1)
    out_ref[...] = lax.dot_general(att, wo_ref[...], dn,
                                   preferred_element_type=jnp.float32) + bo_ref[...]


@jax.jit
def _run(x, edge_index, Wq, bq, Wk, bk, Wv, bv, Wo, bo):
    interpret = False
    edge_flat = edge_index.reshape(2 * E)
    zeros_scat = jnp.zeros((EPW,), jnp.float32)
    mref = jax.new_ref(jnp.full((N * N,), NEG, jnp.float32))
    _build_mask(edge_flat, zeros_scat, mref)
    mask = mref[...].reshape(N, N)

    bk2 = bk.reshape(1, D)
    bv2 = bv.reshape(1, D)
    bq2 = bq.reshape(1, D)
    bo2 = bo.reshape(1, D)

    full = lambda i: (0, 0)
    kv = pl.pallas_call(
        _kv_proj_body,
        grid=(N // BKV,),
        in_specs=[
            pl.BlockSpec((BKV, D), lambda i: (i, 0)),
            pl.BlockSpec((D, D), full),
            pl.BlockSpec((1, D), full),
            pl.BlockSpec((D, D), full),
            pl.BlockSpec((1, D), full),
        ],
        out_specs=[
            pl.BlockSpec((BKV, D), lambda i: (i, 0)),
            pl.BlockSpec((BKV, D), lambda i: (i, 0)),
        ],
        out_shape=[
            jax.ShapeDtypeStruct((N, D), jnp.bfloat16),
            jax.ShapeDtypeStruct((N, D), jnp.bfloat16),
        ],
        interpret=interpret,
    )
    k, v = kv(x, Wk, bk2, Wv, bv2)

    attn = pl.pallas_call(
        _attn_body,
        grid=(N // BQ,),
        in_specs=[
            pl.BlockSpec((BQ, D), lambda i: (i, 0)),
            pl.BlockSpec((D, D), full),
            pl.BlockSpec((1, D), full),
            pl.BlockSpec((N, D), full),
            pl.BlockSpec((N, D), full),
            pl.BlockSpec((BQ, N), lambda i: (i, 0)),
            pl.BlockSpec((D, D), full),
            pl.BlockSpec((1, D), full),
        ],
        out_specs=pl.BlockSpec((BQ, D), lambda i: (i, 0)),
        out_shape=jax.ShapeDtypeStruct((N, D), jnp.float32),
        interpret=interpret,
    )
    return attn(x, Wq, bq2, k, v, mask, Wo, bo2)


def kernel(x, edge_index, Wq, bq, Wk, bk, Wv, bv, Wo, bo):
    return _run(x, edge_index, Wq, bq, Wk, bk, Wv, bv, Wo, bo)


# speedup vs baseline: 1.0053x; 1.0053x over previous
import functools
import math

import jax
import jax.numpy as jnp
from jax import lax
from jax.experimental import pallas as pl
from jax.experimental.pallas import tpu as pltpu
from jax.experimental.pallas import tpu_sc as plsc

N = 4096
D = 512
H = 4
HD = D // H
E = 131072
BQ = 256
BKV = 256
NEG = -1e30

SC_CORES = 2
SC_TILES = 16
EPW = E // (SC_CORES * SC_TILES)


def _scatter_body(edge_ref, ones_ref, mask_ref, rbuf, cbuf, idx1d, ones1d,
                  sem, sem2):
    core = lax.axis_index("c")
    sub = lax.axis_index("s")
    wid = sub * SC_CORES + core
    h_r = pltpu.async_copy(edge_ref.at[pl.ds(wid * EPW, EPW)], rbuf, sem2)
    h_c = pltpu.async_copy(edge_ref.at[pl.ds(E + wid * EPW, EPW)], cbuf, sem2)
    pltpu.sync_copy(ones_ref, ones1d)
    h_r.wait()
    h_c.wait()

    def compute_row(j, carry):
        for i in range(8):
            off = j * 128 + i * 16
            rv = rbuf[pl.ds(off, 16)]
            cv = cbuf[pl.ds(off, 16)]
            idx1d[pl.ds(off, 16)] = rv * N + cv
        return carry

    lax.fori_loop(0, EPW // 128, compute_row, 0)

    pltpu.async_copy(ones1d, mask_ref.at[idx1d], sem).wait()


def _build_mask(edge_flat, ones_arr, mask_ref):
    mesh = plsc.VectorSubcoreMesh(core_axis_name="c", subcore_axis_name="s",
                                  num_cores=SC_CORES)
    f = pl.kernel(
        _scatter_body,
        mesh=mesh,
        out_type=(),
        scratch_types=[
            pltpu.VMEM((EPW,), jnp.int32),
            pltpu.VMEM((EPW,), jnp.int32),
            pltpu.VMEM((EPW,), jnp.int32),
            pltpu.VMEM((EPW,), jnp.float32),
            pltpu.SemaphoreType.DMA,
            pltpu.SemaphoreType.DMA,
        ],
    )
    return f(edge_flat, ones_arr, mask_ref)


def _kv_proj_body(x_ref, wk_ref, bk_ref, wv_ref, bv_ref, k_ref, v_ref):
    x = x_ref[...]
    dn = (((1,), (1,)), ((), ()))
    k = lax.dot_general(x, wk_ref[...], dn,
                        preferred_element_type=jnp.float32) + bk_ref[...]
    v = lax.dot_general(x, wv_ref[...], dn,
                        preferred_element_type=jnp.float32) + bv_ref[...]
    k_ref[...] = k.astype(jnp.bfloat16)
    v_ref[...] = v.astype(jnp.bfloat16)


def _attn_body(x_ref, wq_ref, bq_ref, k_ref, v_ref, mask_ref, wo_ref, bo_ref,
               out_ref):
    dn = (((1,), (1,)), ((), ()))
    x = x_ref[...].astype(jnp.bfloat16)
    q = lax.dot_general(x, wq_ref[...].astype(jnp.bfloat16), dn,
                        preferred_element_type=jnp.float32) + bq_ref[...]
    q = q * (1.0 / math.sqrt(HD))
    bias = mask_ref[...]
    heads = []
    for h in range(H):
        sl = slice(h * HD, (h + 1) * HD)
        qh = q[:, sl].astype(jnp.bfloat16)
        kh = k_ref[:, sl]
        s = lax.dot_general(qh, kh, dn, preferred_element_type=jnp.float32)
        s = s + bias
        m = jnp.max(s, axis=1, keepdims=True)
        valid = m > -0.5e30
        e = jnp.exp(s - m)
        l = jnp.sum(e, axis=1, keepdims=True)
        acc = jnp.dot(e.astype(jnp.bfloat16), v_ref[:, sl],
                      preferred_element_type=jnp.float32)
        heads.append(jnp.where(valid, acc / jnp.maximum(l, 1e-30), 0.0))
    att = jnp.concatenate(heads, axis=1)
    out_ref[...] = lax.dot_general(att, wo_ref[...], dn,
                                   preferred_element_type=jnp.float32) + bo_ref[...]


@jax.jit
def _run(x, edge_index, Wq, bq, Wk, bk, Wv, bv, Wo, bo):
    interpret = False
    edge_flat = edge_index.reshape(2 * E)
    zeros_scat = jnp.zeros((EPW,), jnp.float32)
    mref = jax.new_ref(jnp.full((N * N,), NEG, jnp.float32))
    _build_mask(edge_flat, zeros_scat, mref)
    mask = mref[...].reshape(N, N)

    bk2 = bk.reshape(1, D)
    bv2 = bv.reshape(1, D)
    bq2 = bq.reshape(1, D)
    bo2 = bo.reshape(1, D)

    full = lambda i: (0, 0)
    kv = pl.pallas_call(
        _kv_proj_body,
        grid=(N // BKV,),
        in_specs=[
            pl.BlockSpec((BKV, D), lambda i: (i, 0)),
            pl.BlockSpec((D, D), full),
            pl.BlockSpec((1, D), full),
            pl.BlockSpec((D, D), full),
            pl.BlockSpec((1, D), full),
        ],
        out_specs=[
            pl.BlockSpec((BKV, D), lambda i: (i, 0)),
            pl.BlockSpec((BKV, D), lambda i: (i, 0)),
        ],
        out_shape=[
            jax.ShapeDtypeStruct((N, D), jnp.bfloat16),
            jax.ShapeDtypeStruct((N, D), jnp.bfloat16),
        ],
        interpret=interpret,
    )
    k, v = kv(x, Wk, bk2, Wv, bv2)

    attn = pl.pallas_call(
        _attn_body,
        grid=(N // BQ,),
        in_specs=[
            pl.BlockSpec((BQ, D), lambda i: (i, 0)),
            pl.BlockSpec((D, D), full),
            pl.BlockSpec((1, D), full),
            pl.BlockSpec((N, D), full),
            pl.BlockSpec((N, D), full),
            pl.BlockSpec((BQ, N), lambda i: (i, 0)),
            pl.BlockSpec((D, D), full),
            pl.BlockSpec((1, D), full),
        ],
        out_specs=pl.BlockSpec((BQ, D), lambda i: (i, 0)),
        out_shape=jax.ShapeDtypeStruct((N, D), jnp.float32),
        interpret=interpret,
    )
    return attn(x, Wq, bq2, k, v, mask, Wo, bo2)


def kernel(x, edge_index, Wq, bq, Wk, bk, Wv, bv, Wo, bo):
    return _run(x, edge_index, Wq, bq, Wk, bk, Wv, bv, Wo, bo)
